# Initial kernel scaffold; baseline (speedup 1.0000x reference)
import functools

import jax
import jax.numpy as jnp
from jax import lax
from jax.experimental import pallas as pl
from jax.experimental.pallas import tpu as pltpu
from jax.experimental.pallas import tpu_sc as plsc

_B, _P, _L = 1024, 20, 8
_NHID = 128
_IMG_FEA = 512
_META_VOCAB = 10000
_REG = 0.001

_NW = 32
_IMG_N = 3 * _B
_IMG_PER = _IMG_N // _NW
_NT = 2 * _B * _P
_TASKS_PER = _NT // _NW
_CH = 16
_NCHUNK = _TASKS_PER // _CH


@functools.partial(
    pl.kernel,
    mesh=plsc.VectorSubcoreMesh(core_axis_name="c", subcore_axis_name="s"),
    out_type=[
        jax.ShapeDtypeStruct((_IMG_N, _IMG_FEA), jnp.float32),
        jax.ShapeDtypeStruct((_NT, _NHID), jnp.float32),
    ],
    scratch_types=[
        pltpu.VMEM((_IMG_PER,), jnp.int32),
        pltpu.VMEM((_IMG_PER, _IMG_FEA), jnp.float32),
        pltpu.VMEM((_CH * _L,), jnp.int32),
        pltpu.VMEM((_CH * _L, _NHID), jnp.float32),
        pltpu.VMEM((_CH * _L,), jnp.float32),
        pltpu.VMEM((_CH, _NHID), jnp.float32),
        pltpu.SemaphoreType.DMA,
    ],
)
def _sc_gather_combine(img_feat, img_ids, table, paths, masks,
                       img_out, res_out,
                       iidx_v, irows_v, idx_v, rows_v, m_v, out_v, sem):
    wid = lax.axis_index("s") * 2 + lax.axis_index("c")

    ibase = wid * _IMG_PER
    pltpu.sync_copy(img_ids.at[pl.ds(ibase, _IMG_PER)], iidx_v)
    pltpu.async_copy(img_feat.at[iidx_v], irows_v, sem).wait()
    pltpu.sync_copy(irows_v, img_out.at[pl.ds(ibase, _IMG_PER)])

    tbase = wid * _TASKS_PER
    ones = jnp.ones((16,), jnp.float32)

    def task_body(t, carry):
        mbase = t * _L
        ms = []
        oms = []
        for l in range(_L):
            sidx = jnp.full((16,), mbase + l, jnp.int32)
            m = plsc.load_gather(m_v, [sidx])
            ms.append(m)
            oms.append(ones - m)
        for c in range(_NHID // 16):
            pe = []
            for l in range(_L):
                tl = rows_v[t * _L + l, pl.ds(c * 16, 16)]
                pe.append(tl * ms[l] + oms[l])
            a0 = pe[0] + ms[1] * pe[1]
            a1 = pe[2] + ms[3] * pe[3]
            a2 = pe[4] + ms[5] * pe[5]
            a3 = pe[6] + ms[7] * pe[7]
            out_v[t, pl.ds(c * 16, 16)] = a0 * a1 + a1 * a2 + a2 * a3
        return carry

    def chunk_body(g, carry):
        base = tbase + g * _CH
        pltpu.sync_copy(paths.at[pl.ds(base * _L, _CH * _L)], idx_v)
        pltpu.sync_copy(masks.at[pl.ds(base * _L, _CH * _L)], m_v)
        pltpu.async_copy(table.at[idx_v], rows_v, sem).wait()
        lax.fori_loop(0, _CH, task_body, 0)
        pltpu.sync_copy(out_v, res_out.at[pl.ds(base, _CH)])
        return carry

    lax.fori_loop(0, _NCHUNK, chunk_body, 0)


_BB = 128
_GRID = _B // _BB
_TABF = _META_VOCAB * _NHID // _GRID


def _tc_body(qr, pr, nr, w, b2, hw, hb2, pres, nres, pleaf, nleaf, tabf,
             loss_ref, sq_ref, sp_ref, sn_ref, st_ref):
    i = pl.program_id(0)
    W = w[...]
    dn = (((1,), (1,)), ((), ()))
    q = lax.dot_general(qr[...], W, dn, preferred_element_type=jnp.float32) + b2[...]
    pI = lax.dot_general(pr[...], W, dn, preferred_element_type=jnp.float32) + b2[...]
    nI = lax.dot_general(nr[...], W, dn, preferred_element_type=jnp.float32) + b2[...]
    hwv = hw[...]
    hb = hb2[0, 0]

    def side(res_ref, leaf_ref, item):
        r = res_ref[0]
        ssq = jnp.sum(r * r, axis=-1, keepdims=True)
        pe = r / jnp.maximum(jnp.sqrt(ssq), 1e-12)
        uim = q * item
        uis = q - item
        amp = jnp.sum(uim * hwv, axis=-1)
        v = uis * hwv
        wgt = amp[:, None] - jnp.sum(v[:, None, :] * pe, axis=-1) + hb
        wgt = wgt * jax.nn.sigmoid(leaf_ref[...] * 2.0)
        wgt = wgt - jnp.max(wgt, axis=-1, keepdims=True)
        e = jnp.exp(wgt)
        wsm = e / jnp.sum(e, axis=-1, keepdims=True)
        pool = jnp.sum(pe * wsm[..., None], axis=1)
        return jnp.sum(q * item + (item - q) * pool, axis=-1)

    ps = side(pres, pleaf, pI)
    ns = side(nres, nleaf, nI)
    part = jnp.sum(jnp.log1p(jnp.exp(ns - ps)))
    tb = tabf[...]

    @pl.when(i == 0)
    def _():
        loss_ref[0, 0] = 0.0
        sq_ref[0, 0] = 0.0
        sp_ref[0, 0] = 0.0
        sn_ref[0, 0] = 0.0
        st_ref[0, 0] = 0.0

    loss_ref[0, 0] += part
    sq_ref[0, 0] += jnp.sum(q * q)
    sp_ref[0, 0] += jnp.sum(pI * pI)
    sn_ref[0, 0] += jnp.sum(nI * nI)
    st_ref[0, 0] += jnp.sum(tb * tb)


_SCALAR = jax.ShapeDtypeStruct((1, 1), jnp.float32)

_tc_finish = pl.pallas_call(
    _tc_body,
    grid=(_GRID,),
    in_specs=[
        pl.BlockSpec((_BB, _IMG_FEA), lambda i: (i, 0)),
        pl.BlockSpec((_BB, _IMG_FEA), lambda i: (i + _GRID, 0)),
        pl.BlockSpec((_BB, _IMG_FEA), lambda i: (i + 2 * _GRID, 0)),
        pl.BlockSpec((_NHID, _IMG_FEA), lambda i: (0, 0)),
        pl.BlockSpec((1, _NHID), lambda i: (0, 0)),
        pl.BlockSpec((1, _NHID), lambda i: (0, 0)),
        pl.BlockSpec((1, 1), lambda i: (0, 0)),
        pl.BlockSpec((1, _BB, _P, _NHID), lambda i: (0, i, 0, 0)),
        pl.BlockSpec((1, _BB, _P, _NHID), lambda i: (1, i, 0, 0)),
        pl.BlockSpec((_BB, _P), lambda i: (i, 0)),
        pl.BlockSpec((_BB, _P), lambda i: (i, 0)),
        pl.BlockSpec((1, _TABF), lambda i: (0, i)),
    ],
    out_specs=[pl.BlockSpec((1, 1), lambda i: (0, 0))] * 5,
    out_shape=[_SCALAR] * 5,
)


def kernel(qry_id, pos_id, neg_id, pos_path, pos_mask, pos_leafnodeMask,
           neg_path, neg_mask, neg_leafnodeMask, img_features, imageW_w,
           imageW_b, meta_table, h_att_w, h_att_b):
    f32 = jnp.float32
    ids = jnp.concatenate([qry_id, pos_id, neg_id], axis=0)[:, 0].astype(jnp.int32)
    paths = jnp.concatenate(
        [pos_path.reshape(-1), neg_path.reshape(-1)]).astype(jnp.int32)
    masks = jnp.concatenate(
        [pos_mask.reshape(-1), neg_mask.reshape(-1)]).astype(f32)

    img_rows, res = _sc_gather_combine(
        img_features.astype(f32), ids, meta_table.astype(f32), paths, masks)

    res4 = res.reshape(2, _B, _P, _NHID)
    loss, sq, sp, sn, st = _tc_finish(
        img_rows, img_rows, img_rows,
        imageW_w.astype(f32), imageW_b.reshape(1, _NHID).astype(f32),
        h_att_w.astype(f32), h_att_b.reshape(1, 1).astype(f32),
        res4, res4,
        pos_leafnodeMask.astype(f32), neg_leafnodeMask.astype(f32),
        meta_table.reshape(1, -1).astype(f32))

    return (loss[0, 0] + _REG * (jnp.sqrt(st[0, 0]) + jnp.sqrt(sq[0, 0])
                                 + jnp.sqrt(sp[0, 0]) + jnp.sqrt(sn[0, 0])))


# speedup vs baseline: 3.3113x; 3.3113x over previous
import functools

import jax
import jax.numpy as jnp
from jax import lax
from jax.experimental import pallas as pl
from jax.experimental.pallas import tpu as pltpu
from jax.experimental.pallas import tpu_sc as plsc

_B, _P, _L = 1024, 20, 8
_NHID = 128
_IMG_FEA = 512
_META_VOCAB = 10000
_REG = 0.001

_NW = 32
_IMG_N = 3 * _B
_IMG_PER = _IMG_N // _NW
_NT = 2 * _B * _P
_TASKS_PER = _NT // _NW
_CH = 16
_NCHUNK = _TASKS_PER // _CH


def _sc_body(img_feat, img_ids, table, paths, masks,
                       img_out, res_out,
                       iidx_v, irows_v, idx_v, rows_v, m_v, out_v, sem):
    wid = lax.axis_index("s") * 2 + lax.axis_index("c")

    ibase = wid * _IMG_PER
    pltpu.sync_copy(img_ids.at[pl.ds(ibase, _IMG_PER)], iidx_v)
    pltpu.async_copy(img_feat.at[iidx_v], irows_v, sem).wait()
    pltpu.sync_copy(irows_v, img_out.at[pl.ds(ibase, _IMG_PER)])

    tbase = wid * _TASKS_PER
    ones = jnp.ones((16,), jnp.float32)

    def task_body(tp, carry):
        mv = m_v[pl.ds(tp * 16, 16)]
        for half in range(2):
            t = tp * 2 + half
            ms = []
            oms = []
            for l in range(_L):
                m = jnp.full((16,), mv[half * _L + l], jnp.float32)
                ms.append(m)
                oms.append(ones - m)
            for c in range(_NHID // 16):
                pe = []
                for l in range(_L):
                    tl = rows_v[t * _L + l, pl.ds(c * 16, 16)]
                    pe.append(tl * ms[l] + oms[l])
                a0 = pe[0] + ms[1] * pe[1]
                a1 = pe[2] + ms[3] * pe[3]
                a2 = pe[4] + ms[5] * pe[5]
                a3 = pe[6] + ms[7] * pe[7]
                out_v[t, pl.ds(c * 16, 16)] = a0 * a1 + a1 * a2 + a2 * a3
        return carry

    def chunk_body(g, carry):
        base = tbase + g * _CH
        pltpu.sync_copy(paths.at[pl.ds(base * _L, _CH * _L)], idx_v)
        pltpu.sync_copy(masks.at[pl.ds(base * _L, _CH * _L)], m_v)
        pltpu.async_copy(table.at[idx_v], rows_v, sem).wait()
        lax.fori_loop(0, _CH // 2, task_body, 0)
        pltpu.sync_copy(out_v, res_out.at[pl.ds(base, _CH)])
        return carry

    lax.fori_loop(0, _NCHUNK, chunk_body, 0)


@functools.cache
def _sc_gather_combine():
    return pl.kernel(
        _sc_body,
        mesh=plsc.VectorSubcoreMesh(core_axis_name="c", subcore_axis_name="s"),
        out_type=[
            jax.ShapeDtypeStruct((_IMG_N, _IMG_FEA), jnp.float32),
            jax.ShapeDtypeStruct((_NT, _NHID), jnp.float32),
        ],
        scratch_types=[
            pltpu.VMEM((_IMG_PER,), jnp.int32),
            pltpu.VMEM((_IMG_PER, _IMG_FEA), jnp.float32),
            pltpu.VMEM((_CH * _L,), jnp.int32),
            pltpu.VMEM((_CH * _L, _NHID), jnp.float32),
            pltpu.VMEM((_CH * _L,), jnp.float32),
            pltpu.VMEM((_CH, _NHID), jnp.float32),
            pltpu.SemaphoreType.DMA,
        ],
    )


_BB = 128
_GRID = _B // _BB
_TABF = _META_VOCAB * _NHID // _GRID


def _tc_body(qr, pr, nr, w, b2, hw, hb2, pres, nres, pleaf, nleaf, tabf,
             loss_ref, sq_ref, sp_ref, sn_ref, st_ref):
    i = pl.program_id(0)
    W = w[...]
    dn = (((1,), (1,)), ((), ()))
    q = lax.dot_general(qr[...], W, dn, preferred_element_type=jnp.float32) + b2[...]
    pI = lax.dot_general(pr[...], W, dn, preferred_element_type=jnp.float32) + b2[...]
    nI = lax.dot_general(nr[...], W, dn, preferred_element_type=jnp.float32) + b2[...]
    hwv = hw[...]
    hb = hb2[0, 0]

    def side(res_ref, leaf_ref, item):
        r = res_ref[0]
        ssq = jnp.sum(r * r, axis=-1, keepdims=True)
        pe = r / jnp.maximum(jnp.sqrt(ssq), 1e-12)
        uim = q * item
        uis = q - item
        amp = jnp.sum(uim * hwv, axis=-1)
        v = uis * hwv
        wgt = amp[:, None] - jnp.sum(v[:, None, :] * pe, axis=-1) + hb
        wgt = wgt * jax.nn.sigmoid(leaf_ref[...] * 2.0)
        wgt = wgt - jnp.max(wgt, axis=-1, keepdims=True)
        e = jnp.exp(wgt)
        wsm = e / jnp.sum(e, axis=-1, keepdims=True)
        pool = jnp.sum(pe * wsm[..., None], axis=1)
        return jnp.sum(q * item + (item - q) * pool, axis=-1)

    ps = side(pres, pleaf, pI)
    ns = side(nres, nleaf, nI)
    part = jnp.sum(jnp.log1p(jnp.exp(ns - ps)))
    tb = tabf[...]

    @pl.when(i == 0)
    def _():
        zero = jnp.zeros((1, 1), jnp.float32)
        loss_ref[...] = zero
        sq_ref[...] = zero
        sp_ref[...] = zero
        sn_ref[...] = zero
        st_ref[...] = zero

    loss_ref[...] += jnp.reshape(part, (1, 1))
    sq_ref[...] += jnp.reshape(jnp.sum(q * q), (1, 1))
    sp_ref[...] += jnp.reshape(jnp.sum(pI * pI), (1, 1))
    sn_ref[...] += jnp.reshape(jnp.sum(nI * nI), (1, 1))
    st_ref[...] += jnp.reshape(jnp.sum(tb * tb), (1, 1))


_SCALAR = jax.ShapeDtypeStruct((1, 1), jnp.float32)

_TC_IN_SPECS = [
        pl.BlockSpec((_BB, _IMG_FEA), lambda i: (i, 0)),
        pl.BlockSpec((_BB, _IMG_FEA), lambda i: (i + _GRID, 0)),
        pl.BlockSpec((_BB, _IMG_FEA), lambda i: (i + 2 * _GRID, 0)),
        pl.BlockSpec((_NHID, _IMG_FEA), lambda i: (0, 0)),
        pl.BlockSpec((1, _NHID), lambda i: (0, 0)),
        pl.BlockSpec((1, _NHID), lambda i: (0, 0)),
        pl.BlockSpec((1, 1), lambda i: (0, 0)),
        pl.BlockSpec((1, _BB, _P, _NHID), lambda i: (0, i, 0, 0)),
        pl.BlockSpec((1, _BB, _P, _NHID), lambda i: (1, i, 0, 0)),
        pl.BlockSpec((_BB, _P), lambda i: (i, 0)),
        pl.BlockSpec((_BB, _P), lambda i: (i, 0)),
        pl.BlockSpec((1, _TABF), lambda i: (0, i)),
]

_tc_finish = pl.pallas_call(
    _tc_body,
    grid=(_GRID,),
    in_specs=_TC_IN_SPECS,
    out_specs=[pl.BlockSpec((1, 1), lambda i: (0, 0))] * 5,
    out_shape=[_SCALAR] * 5,
)


def kernel(qry_id, pos_id, neg_id, pos_path, pos_mask, pos_leafnodeMask,
           neg_path, neg_mask, neg_leafnodeMask, img_features, imageW_w,
           imageW_b, meta_table, h_att_w, h_att_b):
    f32 = jnp.float32
    ids = jnp.concatenate([qry_id, pos_id, neg_id], axis=0)[:, 0].astype(jnp.int32)
    paths = jnp.concatenate(
        [pos_path.reshape(-1), neg_path.reshape(-1)]).astype(jnp.int32)
    masks = jnp.concatenate(
        [pos_mask.reshape(-1), neg_mask.reshape(-1)]).astype(f32)

    img_rows, res = _sc_gather_combine()(
        img_features.astype(f32), ids, meta_table.astype(f32), paths, masks)

    res4 = res.reshape(2, _B, _P, _NHID)
    loss, sq, sp, sn, st = _tc_finish(
        img_rows, img_rows, img_rows,
        imageW_w.astype(f32), imageW_b.reshape(1, _NHID).astype(f32),
        h_att_w.astype(f32), h_att_b.reshape(1, 1).astype(f32),
        res4, res4,
        pos_leafnodeMask.astype(f32), neg_leafnodeMask.astype(f32),
        meta_table.reshape(1, -1).astype(f32))

    return (loss[0, 0] + _REG * (jnp.sqrt(st[0, 0]) + jnp.sqrt(sq[0, 0])
                                 + jnp.sqrt(sp[0, 0]) + jnp.sqrt(sn[0, 0])))


# speedup vs baseline: 4.9995x; 1.5098x over previous
import functools

import jax
import jax.numpy as jnp
from jax import lax
from jax.experimental import pallas as pl
from jax.experimental.pallas import tpu as pltpu
from jax.experimental.pallas import tpu_sc as plsc

_B, _P, _L = 1024, 20, 8
_NHID = 128
_IMG_FEA = 512
_META_VOCAB = 10000
_REG = 0.001

_NW = 32
_IMG_N = 3 * _B
_IMG_PER = _IMG_N // _NW
_NT = 2 * _B * _P
_TASKS_PER = _NT // _NW
_CH = 32
_NCHUNK = _TASKS_PER // _CH


def _sc_body(img_feat, img_ids, table, paths, masks,
                       img_out, res_out,
                       iidx_v, irows_v, idx_v, rows_v, m_v, out_v, sem, osem):
    wid = lax.axis_index("s") * 2 + lax.axis_index("c")

    ibase = wid * _IMG_PER
    pltpu.sync_copy(img_ids.at[pl.ds(ibase, _IMG_PER)], iidx_v)
    for h in range(2):
        pltpu.async_copy(
            img_feat.at[iidx_v.at[pl.ds(h * _IMG_PER // 2, _IMG_PER // 2)]],
            irows_v, sem).wait()
        pltpu.sync_copy(
            irows_v, img_out.at[pl.ds(ibase + h * _IMG_PER // 2, _IMG_PER // 2)])

    tbase = wid * _TASKS_PER
    ones = jnp.ones((16,), jnp.float32)
    _R = _CH * _L

    pltpu.sync_copy(masks.at[pl.ds(tbase * _L, _TASKS_PER * _L)], m_v)

    def fire(g, slot):
        base = tbase + g * _CH
        pltpu.sync_copy(paths.at[pl.ds(base * _L, _R)],
                        idx_v.at[pl.ds(slot * _R, _R)])
        for h in range(2):
            pltpu.async_copy(
                table.at[idx_v.at[pl.ds(slot * _R + h * 128, 128)]],
                rows_v.at[pl.ds(slot * _R + h * 128, 128)], sem)

    def task_body(tp, soff):
        roff, ooff, moff = soff
        mv = m_v[pl.ds(moff + tp * 16, 16)]
        for half in range(2):
            t = tp * 2 + half
            ms = []
            oms = []
            for l in range(_L):
                m = jnp.full((16,), mv[half * _L + l], jnp.float32)
                ms.append(m)
                oms.append(ones - m)
            for c in range(_NHID // 16):
                pe = []
                for l in range(_L):
                    tl = rows_v[roff + t * _L + l, pl.ds(c * 16, 16)]
                    pe.append(tl * ms[l] + oms[l])
                a0 = pe[0] + ms[1] * pe[1]
                a1 = pe[2] + ms[3] * pe[3]
                a2 = pe[4] + ms[5] * pe[5]
                a3 = pe[6] + ms[7] * pe[7]
                out_v[ooff + t, pl.ds(c * 16, 16)] = a0 * a1 + a1 * a2 + a2 * a3
        return soff

    fire(0, 0)

    def chunk_body(g, carry):
        slot = lax.rem(g, 2)
        nslot = 1 - slot
        base = tbase + g * _CH

        @pl.when(g + 1 < _NCHUNK)
        def _():
            fire(g + 1, nslot)

        @pl.when(g >= 2)
        def _():
            pltpu.make_async_copy(
                out_v.at[pl.ds(slot * _CH, _CH)],
                res_out.at[pl.ds(base, _CH)], osem).wait()

        for h in range(2):
            pltpu.make_async_copy(
                table.at[idx_v.at[pl.ds(slot * _R + h * 128, 128)]],
                rows_v.at[pl.ds(slot * _R + h * 128, 128)], sem).wait()

        lax.fori_loop(0, _CH // 2, task_body,
                      (slot * _R, slot * _CH, g * _CH * _L))
        pltpu.async_copy(out_v.at[pl.ds(slot * _CH, _CH)],
                         res_out.at[pl.ds(base, _CH)], osem)
        return carry

    lax.fori_loop(0, _NCHUNK, chunk_body, 0)

    for k in range(2):
        pltpu.make_async_copy(
            out_v.at[pl.ds(k * _CH, _CH)],
            res_out.at[pl.ds(tbase + k * _CH, _CH)], osem).wait()


@functools.cache
def _sc_gather_combine():
    return pl.kernel(
        _sc_body,
        mesh=plsc.VectorSubcoreMesh(core_axis_name="c", subcore_axis_name="s"),
        out_type=[
            jax.ShapeDtypeStruct((_IMG_N, _IMG_FEA), jnp.float32),
            jax.ShapeDtypeStruct((_NT, _NHID), jnp.float32),
        ],
        scratch_types=[
            pltpu.VMEM((_IMG_PER,), jnp.int32),
            pltpu.VMEM((_IMG_PER // 2, _IMG_FEA), jnp.float32),
            pltpu.VMEM((2 * _CH * _L,), jnp.int32),
            pltpu.VMEM((2 * _CH * _L, _NHID), jnp.float32),
            pltpu.VMEM((_TASKS_PER * _L,), jnp.float32),
            pltpu.VMEM((2 * _CH, _NHID), jnp.float32),
            pltpu.SemaphoreType.DMA,
            pltpu.SemaphoreType.DMA,
        ],
    )


_BB = 128
_GRID = _B // _BB
_TABF = _META_VOCAB * _NHID // _GRID


def _tc_body(qr, pr, nr, w, b2, hw, hb2, pres, nres, pleaf, nleaf, tabf,
             loss_ref, sq_ref, sp_ref, sn_ref, st_ref):
    i = pl.program_id(0)
    W = w[...]
    dn = (((1,), (1,)), ((), ()))
    q = lax.dot_general(qr[...], W, dn, preferred_element_type=jnp.float32) + b2[...]
    pI = lax.dot_general(pr[...], W, dn, preferred_element_type=jnp.float32) + b2[...]
    nI = lax.dot_general(nr[...], W, dn, preferred_element_type=jnp.float32) + b2[...]
    hwv = hw[...]
    hb = hb2[0, 0]

    def side(res_ref, leaf_ref, item):
        r = res_ref[0]
        ssq = jnp.sum(r * r, axis=-1, keepdims=True)
        pe = r / jnp.maximum(jnp.sqrt(ssq), 1e-12)
        uim = q * item
        uis = q - item
        amp = jnp.sum(uim * hwv, axis=-1)
        v = uis * hwv
        wgt = amp[:, None] - jnp.sum(v[:, None, :] * pe, axis=-1) + hb
        wgt = wgt * jax.nn.sigmoid(leaf_ref[...] * 2.0)
        wgt = wgt - jnp.max(wgt, axis=-1, keepdims=True)
        e = jnp.exp(wgt)
        wsm = e / jnp.sum(e, axis=-1, keepdims=True)
        pool = jnp.sum(pe * wsm[..., None], axis=1)
        return jnp.sum(q * item + (item - q) * pool, axis=-1)

    ps = side(pres, pleaf, pI)
    ns = side(nres, nleaf, nI)
    part = jnp.sum(jnp.log1p(jnp.exp(ns - ps)))
    tb = tabf[...]

    @pl.when(i == 0)
    def _():
        zero = jnp.zeros((1, 1), jnp.float32)
        loss_ref[...] = zero
        sq_ref[...] = zero
        sp_ref[...] = zero
        sn_ref[...] = zero
        st_ref[...] = zero

    loss_ref[...] += jnp.reshape(part, (1, 1))
    sq_ref[...] += jnp.reshape(jnp.sum(q * q), (1, 1))
    sp_ref[...] += jnp.reshape(jnp.sum(pI * pI), (1, 1))
    sn_ref[...] += jnp.reshape(jnp.sum(nI * nI), (1, 1))
    st_ref[...] += jnp.reshape(jnp.sum(tb * tb), (1, 1))


_SCALAR = jax.ShapeDtypeStruct((1, 1), jnp.float32)

_TC_IN_SPECS = [
        pl.BlockSpec((_BB, _IMG_FEA), lambda i: (i, 0)),
        pl.BlockSpec((_BB, _IMG_FEA), lambda i: (i + _GRID, 0)),
        pl.BlockSpec((_BB, _IMG_FEA), lambda i: (i + 2 * _GRID, 0)),
        pl.BlockSpec((_NHID, _IMG_FEA), lambda i: (0, 0)),
        pl.BlockSpec((1, _NHID), lambda i: (0, 0)),
        pl.BlockSpec((1, _NHID), lambda i: (0, 0)),
        pl.BlockSpec((1, 1), lambda i: (0, 0)),
        pl.BlockSpec((1, _BB, _P, _NHID), lambda i: (0, i, 0, 0)),
        pl.BlockSpec((1, _BB, _P, _NHID), lambda i: (1, i, 0, 0)),
        pl.BlockSpec((_BB, _P), lambda i: (i, 0)),
        pl.BlockSpec((_BB, _P), lambda i: (i, 0)),
        pl.BlockSpec((1, _TABF), lambda i: (0, i)),
]

_tc_finish = pl.pallas_call(
    _tc_body,
    grid=(_GRID,),
    in_specs=_TC_IN_SPECS,
    out_specs=[pl.BlockSpec((1, 1), lambda i: (0, 0))] * 5,
    out_shape=[_SCALAR] * 5,
)


def kernel(qry_id, pos_id, neg_id, pos_path, pos_mask, pos_leafnodeMask,
           neg_path, neg_mask, neg_leafnodeMask, img_features, imageW_w,
           imageW_b, meta_table, h_att_w, h_att_b):
    f32 = jnp.float32
    ids = jnp.concatenate([qry_id, pos_id, neg_id], axis=0)[:, 0].astype(jnp.int32)
    paths = jnp.concatenate(
        [pos_path.reshape(-1), neg_path.reshape(-1)]).astype(jnp.int32)
    masks = jnp.concatenate(
        [pos_mask.reshape(-1), neg_mask.reshape(-1)]).astype(f32)

    img_rows, res = _sc_gather_combine()(
        img_features.astype(f32), ids, meta_table.astype(f32), paths, masks)

    res4 = res.reshape(2, _B, _P, _NHID)
    loss, sq, sp, sn, st = _tc_finish(
        img_rows, img_rows, img_rows,
        imageW_w.astype(f32), imageW_b.reshape(1, _NHID).astype(f32),
        h_att_w.astype(f32), h_att_b.reshape(1, 1).astype(f32),
        res4, res4,
        pos_leafnodeMask.astype(f32), neg_leafnodeMask.astype(f32),
        meta_table.reshape(1, -1).astype(f32))

    return (loss[0, 0] + _REG * (jnp.sqrt(st[0, 0]) + jnp.sqrt(sq[0, 0])
                                 + jnp.sqrt(sp[0, 0]) + jnp.sqrt(sn[0, 0])))


# speedup vs baseline: 5.4176x; 1.0836x over previous
import functools

import jax
import jax.numpy as jnp
from jax import lax
from jax.experimental import pallas as pl
from jax.experimental.pallas import tpu as pltpu
from jax.experimental.pallas import tpu_sc as plsc

_B, _P, _L = 1024, 20, 8
_NHID = 128
_IMG_FEA = 512
_META_VOCAB = 10000
_REG = 0.001

_NW = 32
_IMG_N = 3 * _B
_IMG_PER = _IMG_N // _NW
_NT = 2 * _B * _P
_TASKS_PER = _NT // _NW
_CH = 32
_NCHUNK = _TASKS_PER // _CH


def _sc_body(img_feat, img_ids, table, paths, masks,
                       img_out, res_out,
                       iidx_v, irows_v, idx_v, rows_v, m_v, out_v, sem, osem):
    wid = lax.axis_index("s") * 2 + lax.axis_index("c")

    ibase = wid * _IMG_PER
    pltpu.sync_copy(img_ids.at[pl.ds(ibase, _IMG_PER)], iidx_v)
    for h in range(2):
        pltpu.async_copy(
            img_feat.at[iidx_v.at[pl.ds(h * _IMG_PER // 2, _IMG_PER // 2)]],
            irows_v, sem).wait()
        pltpu.sync_copy(
            irows_v, img_out.at[pl.ds(ibase + h * _IMG_PER // 2, _IMG_PER // 2)])

    tbase = wid * _TASKS_PER
    ones = jnp.ones((16,), jnp.float32)
    _R = _CH * _L

    pltpu.sync_copy(masks.at[pl.ds(tbase * _L, _TASKS_PER * _L)], m_v)

    def fire(g, slot):
        base = tbase + g * _CH
        pltpu.sync_copy(paths.at[pl.ds(base * _L, _R)],
                        idx_v.at[pl.ds(slot * _R, _R)])
        for h in range(2):
            pltpu.async_copy(
                table.at[idx_v.at[pl.ds(slot * _R + h * 128, 128)]],
                rows_v.at[pl.ds(slot * _R + h * 128, 128)], sem)

    def task_body(tp, soff):
        roff, ooff, moff = soff
        mv = m_v[pl.ds(moff + tp * 16, 16)]
        for half in range(2):
            t = tp * 2 + half
            ms = [jnp.full((16,), mv[half * _L + l], jnp.float32)
                  for l in range(_L)]
            c0 = [ms[2 * i] for i in range(4)]
            c1 = [ms[2 * i + 1] * ms[2 * i + 1] for i in range(4)]
            c2 = [(ones - ms[2 * i]) + (ms[2 * i + 1] - c1[i]) for i in range(4)]
            for c in range(_NHID // 16):
                a = []
                for i in range(4):
                    t0 = rows_v[roff + t * _L + 2 * i, pl.ds(c * 16, 16)]
                    t1 = rows_v[roff + t * _L + 2 * i + 1, pl.ds(c * 16, 16)]
                    a.append(t0 * c0[i] + t1 * c1[i] + c2[i])
                out_v[ooff + t, pl.ds(c * 16, 16)] = (
                    a[1] * (a[0] + a[2]) + a[2] * a[3])
        return soff

    fire(0, 0)

    def chunk_body(g, carry):
        slot = lax.rem(g, 2)
        nslot = 1 - slot
        base = tbase + g * _CH

        @pl.when(g + 1 < _NCHUNK)
        def _():
            fire(g + 1, nslot)

        @pl.when(g >= 2)
        def _():
            pltpu.make_async_copy(
                out_v.at[pl.ds(slot * _CH, _CH)],
                res_out.at[pl.ds(base, _CH)], osem).wait()

        for h in range(2):
            pltpu.make_async_copy(
                table.at[idx_v.at[pl.ds(slot * _R + h * 128, 128)]],
                rows_v.at[pl.ds(slot * _R + h * 128, 128)], sem).wait()

        lax.fori_loop(0, _CH // 2, task_body,
                      (slot * _R, slot * _CH, g * _CH * _L))
        pltpu.async_copy(out_v.at[pl.ds(slot * _CH, _CH)],
                         res_out.at[pl.ds(base, _CH)], osem)
        return carry

    lax.fori_loop(0, _NCHUNK, chunk_body, 0)

    for k in range(2):
        pltpu.make_async_copy(
            out_v.at[pl.ds(k * _CH, _CH)],
            res_out.at[pl.ds(tbase + k * _CH, _CH)], osem).wait()


@functools.cache
def _sc_gather_combine():
    return pl.kernel(
        _sc_body,
        mesh=plsc.VectorSubcoreMesh(core_axis_name="c", subcore_axis_name="s"),
        out_type=[
            jax.ShapeDtypeStruct((_IMG_N, _IMG_FEA), jnp.float32),
            jax.ShapeDtypeStruct((_NT, _NHID), jnp.float32),
        ],
        scratch_types=[
            pltpu.VMEM((_IMG_PER,), jnp.int32),
            pltpu.VMEM((_IMG_PER // 2, _IMG_FEA), jnp.float32),
            pltpu.VMEM((2 * _CH * _L,), jnp.int32),
            pltpu.VMEM((2 * _CH * _L, _NHID), jnp.float32),
            pltpu.VMEM((_TASKS_PER * _L,), jnp.float32),
            pltpu.VMEM((2 * _CH, _NHID), jnp.float32),
            pltpu.SemaphoreType.DMA,
            pltpu.SemaphoreType.DMA,
        ],
    )


_BB = 128
_GRID = _B // _BB
_TABF = _META_VOCAB * _NHID // _GRID


def _tc_body(qr, pr, nr, w, b2, hw, hb2, pres, nres, pleaf, nleaf, tabf,
             loss_ref, sq_ref, sp_ref, sn_ref, st_ref):
    i = pl.program_id(0)
    W = w[...]
    dn = (((1,), (1,)), ((), ()))
    q = lax.dot_general(qr[...], W, dn, preferred_element_type=jnp.float32) + b2[...]
    pI = lax.dot_general(pr[...], W, dn, preferred_element_type=jnp.float32) + b2[...]
    nI = lax.dot_general(nr[...], W, dn, preferred_element_type=jnp.float32) + b2[...]
    hwv = hw[...]
    hb = hb2[0, 0]

    def side(res_ref, leaf_ref, item):
        r = res_ref[...].reshape(_BB, _P, _NHID)
        ssq = jnp.sum(r * r, axis=-1, keepdims=True)
        pe = r / jnp.maximum(jnp.sqrt(ssq), 1e-12)
        uim = q * item
        uis = q - item
        amp = jnp.sum(uim * hwv, axis=-1)
        v = uis * hwv
        wgt = amp[:, None] - jnp.sum(v[:, None, :] * pe, axis=-1) + hb
        wgt = wgt * jax.nn.sigmoid(leaf_ref[...] * 2.0)
        wgt = wgt - jnp.max(wgt, axis=-1, keepdims=True)
        e = jnp.exp(wgt)
        wsm = e / jnp.sum(e, axis=-1, keepdims=True)
        pool = jnp.sum(pe * wsm[..., None], axis=1)
        return jnp.sum(q * item + (item - q) * pool, axis=-1)

    ps = side(pres, pleaf, pI)
    ns = side(nres, nleaf, nI)
    part = jnp.sum(jnp.log1p(jnp.exp(ns - ps)))
    tb = tabf[...]

    @pl.when(i == 0)
    def _():
        zero = jnp.zeros((1, 1), jnp.float32)
        loss_ref[...] = zero
        sq_ref[...] = zero
        sp_ref[...] = zero
        sn_ref[...] = zero
        st_ref[...] = zero

    loss_ref[...] += jnp.reshape(part, (1, 1))
    sq_ref[...] += jnp.reshape(jnp.sum(q * q), (1, 1))
    sp_ref[...] += jnp.reshape(jnp.sum(pI * pI), (1, 1))
    sn_ref[...] += jnp.reshape(jnp.sum(nI * nI), (1, 1))
    st_ref[...] += jnp.reshape(jnp.sum(tb * tb), (1, 1))


_SCALAR = jax.ShapeDtypeStruct((1, 1), jnp.float32)

_TC_IN_SPECS = [
        pl.BlockSpec((_BB, _IMG_FEA), lambda i: (i, 0)),
        pl.BlockSpec((_BB, _IMG_FEA), lambda i: (i + _GRID, 0)),
        pl.BlockSpec((_BB, _IMG_FEA), lambda i: (i + 2 * _GRID, 0)),
        pl.BlockSpec((_NHID, _IMG_FEA), lambda i: (0, 0)),
        pl.BlockSpec((1, _NHID), lambda i: (0, 0)),
        pl.BlockSpec((1, _NHID), lambda i: (0, 0)),
        pl.BlockSpec((1, 1), lambda i: (0, 0)),
        pl.BlockSpec((_BB * _P, _NHID), lambda i: (i, 0)),
        pl.BlockSpec((_BB * _P, _NHID), lambda i: (i + _GRID, 0)),
        pl.BlockSpec((_BB, _P), lambda i: (i, 0)),
        pl.BlockSpec((_BB, _P), lambda i: (i, 0)),
        pl.BlockSpec((1, _TABF), lambda i: (0, i)),
]

_tc_finish = pl.pallas_call(
    _tc_body,
    grid=(_GRID,),
    in_specs=_TC_IN_SPECS,
    out_specs=[pl.BlockSpec((1, 1), lambda i: (0, 0))] * 5,
    out_shape=[_SCALAR] * 5,
)


def kernel(qry_id, pos_id, neg_id, pos_path, pos_mask, pos_leafnodeMask,
           neg_path, neg_mask, neg_leafnodeMask, img_features, imageW_w,
           imageW_b, meta_table, h_att_w, h_att_b):
    f32 = jnp.float32
    ids = jnp.concatenate([qry_id, pos_id, neg_id], axis=0)[:, 0].astype(jnp.int32)
    paths = jnp.concatenate(
        [pos_path.reshape(-1), neg_path.reshape(-1)]).astype(jnp.int32)
    masks = jnp.concatenate(
        [pos_mask.reshape(-1), neg_mask.reshape(-1)]).astype(f32)

    img_rows, res = _sc_gather_combine()(
        img_features.astype(f32), ids, meta_table.astype(f32), paths, masks)

    loss, sq, sp, sn, st = _tc_finish(
        img_rows, img_rows, img_rows,
        imageW_w.astype(f32), imageW_b.reshape(1, _NHID).astype(f32),
        h_att_w.astype(f32), h_att_b.reshape(1, 1).astype(f32),
        res, res,
        pos_leafnodeMask.astype(f32), neg_leafnodeMask.astype(f32),
        meta_table.reshape(1, -1).astype(f32))

    return (loss[0, 0] + _REG * (jnp.sqrt(st[0, 0]) + jnp.sqrt(sq[0, 0])
                                 + jnp.sqrt(sp[0, 0]) + jnp.sqrt(sn[0, 0])))


# speedup vs baseline: 6.7859x; 1.2526x over previous
import functools

import jax
import jax.numpy as jnp
from jax import lax
from jax.experimental import pallas as pl
from jax.experimental.pallas import tpu as pltpu
from jax.experimental.pallas import tpu_sc as plsc

_B, _P, _L = 1024, 20, 8
_NHID = 128
_IMG_FEA = 512
_META_VOCAB = 10000
_REG = 0.001

_NW = 32
_IMG_N = 3 * _B
_IMG_PER = _IMG_N // _NW
_NT = 2 * _B * _P
_TASKS_PER = _NT // _NW
_CH = 32
_NCHUNK = _TASKS_PER // _CH


def _sc_body(img_feat, img_ids, table, paths, masks,
                       img_out, res_out,
                       iidx_v, irows_v, idx_v, rows_v, m_v, out_v, sem, osem):
    wid = lax.axis_index("s") * 2 + lax.axis_index("c")

    ibase = wid * _IMG_PER
    pltpu.sync_copy(img_ids.at[pl.ds(ibase, _IMG_PER)], iidx_v)
    for h in range(2):
        pltpu.async_copy(
            img_feat.at[iidx_v.at[pl.ds(h * _IMG_PER // 2, _IMG_PER // 2)]],
            irows_v, sem).wait()
        pltpu.sync_copy(
            irows_v, img_out.at[pl.ds(ibase + h * _IMG_PER // 2, _IMG_PER // 2)])

    tbase = wid * _TASKS_PER
    ones = jnp.ones((16,), jnp.float32)
    _R = _CH * _L

    pltpu.sync_copy(masks.at[pl.ds(tbase * _L, _TASKS_PER * _L)], m_v)

    def fire(g, slot):
        base = tbase + g * _CH
        pltpu.sync_copy(paths.at[pl.ds(base * _L, _R)],
                        idx_v.at[pl.ds(slot * _R, _R)])
        for h in range(2):
            pltpu.async_copy(
                table.at[idx_v.at[pl.ds(slot * _R + h * 128, 128)]],
                rows_v.at[pl.ds(slot * _R + h * 128, 128)], sem)

    def run_tasks(roff, ooff, moff):
        @plsc.parallel_loop(0, _CH // 2, 1, unroll=2)
        def _(tp):
            mv = m_v[pl.ds(moff + tp * 16, 16)]
            for half in range(2):
                t = tp * 2 + half
                ms = [jnp.full((16,), mv[half * _L + l], jnp.float32)
                      for l in range(_L)]
                c0 = [ms[2 * i] for i in range(4)]
                c1 = [ms[2 * i + 1] * ms[2 * i + 1] for i in range(4)]
                c2 = [(ones - ms[2 * i]) + (ms[2 * i + 1] - c1[i])
                      for i in range(4)]
                for c in range(_NHID // 16):
                    a = []
                    for i in range(4):
                        t0 = rows_v[roff + t * _L + 2 * i, pl.ds(c * 16, 16)]
                        t1 = rows_v[roff + t * _L + 2 * i + 1, pl.ds(c * 16, 16)]
                        a.append(t0 * c0[i] + t1 * c1[i] + c2[i])
                    out_v[ooff + t, pl.ds(c * 16, 16)] = (
                        a[1] * (a[0] + a[2]) + a[2] * a[3])

    fire(0, 0)

    def chunk_body(g, carry):
        slot = lax.rem(g, 2)
        nslot = 1 - slot
        base = tbase + g * _CH

        @pl.when(g + 1 < _NCHUNK)
        def _():
            fire(g + 1, nslot)

        @pl.when(g >= 2)
        def _():
            pltpu.make_async_copy(
                out_v.at[pl.ds(slot * _CH, _CH)],
                res_out.at[pl.ds(base, _CH)], osem).wait()

        for h in range(2):
            pltpu.make_async_copy(
                table.at[idx_v.at[pl.ds(slot * _R + h * 128, 128)]],
                rows_v.at[pl.ds(slot * _R + h * 128, 128)], sem).wait()

        run_tasks(slot * _R, slot * _CH, g * _CH * _L)
        pltpu.async_copy(out_v.at[pl.ds(slot * _CH, _CH)],
                         res_out.at[pl.ds(base, _CH)], osem)
        return carry

    lax.fori_loop(0, _NCHUNK, chunk_body, 0)

    for k in range(2):
        pltpu.make_async_copy(
            out_v.at[pl.ds(k * _CH, _CH)],
            res_out.at[pl.ds(tbase + k * _CH, _CH)], osem).wait()


@functools.cache
def _sc_gather_combine():
    return pl.kernel(
        _sc_body,
        mesh=plsc.VectorSubcoreMesh(core_axis_name="c", subcore_axis_name="s"),
        out_type=[
            jax.ShapeDtypeStruct((_IMG_N, _IMG_FEA), jnp.float32),
            jax.ShapeDtypeStruct((_NT, _NHID), jnp.float32),
        ],
        scratch_types=[
            pltpu.VMEM((_IMG_PER,), jnp.int32),
            pltpu.VMEM((_IMG_PER // 2, _IMG_FEA), jnp.float32),
            pltpu.VMEM((2 * _CH * _L,), jnp.int32),
            pltpu.VMEM((2 * _CH * _L, _NHID), jnp.float32),
            pltpu.VMEM((_TASKS_PER * _L,), jnp.float32),
            pltpu.VMEM((2 * _CH, _NHID), jnp.float32),
            pltpu.SemaphoreType.DMA,
            pltpu.SemaphoreType.DMA,
        ],
    )


_BB = 128
_GRID = _B // _BB
_TABF = _META_VOCAB * _NHID // _GRID


def _tc_body(qr, pr, nr, w, b2, hw, hb2, pres, nres, pleaf, nleaf, tabf,
             loss_ref, sq_ref, sp_ref, sn_ref, st_ref):
    i = pl.program_id(0)
    W = w[...]
    dn = (((1,), (1,)), ((), ()))
    q = lax.dot_general(qr[...], W, dn, preferred_element_type=jnp.float32) + b2[...]
    pI = lax.dot_general(pr[...], W, dn, preferred_element_type=jnp.float32) + b2[...]
    nI = lax.dot_general(nr[...], W, dn, preferred_element_type=jnp.float32) + b2[...]
    hwv = hw[...]
    hb = hb2[0, 0]

    def side(res_ref, leaf_ref, item):
        r = res_ref[...].reshape(_BB, _P, _NHID)
        ssq = jnp.sum(r * r, axis=-1, keepdims=True)
        pe = r / jnp.maximum(jnp.sqrt(ssq), 1e-12)
        uim = q * item
        uis = q - item
        amp = jnp.sum(uim * hwv, axis=-1)
        v = uis * hwv
        wgt = amp[:, None] - jnp.sum(v[:, None, :] * pe, axis=-1) + hb
        wgt = wgt * jax.nn.sigmoid(leaf_ref[...] * 2.0)
        wgt = wgt - jnp.max(wgt, axis=-1, keepdims=True)
        e = jnp.exp(wgt)
        wsm = e / jnp.sum(e, axis=-1, keepdims=True)
        pool = jnp.sum(pe * wsm[..., None], axis=1)
        return jnp.sum(q * item + (item - q) * pool, axis=-1)

    ps = side(pres, pleaf, pI)
    ns = side(nres, nleaf, nI)
    part = jnp.sum(jnp.log1p(jnp.exp(ns - ps)))
    tb = tabf[...]

    @pl.when(i == 0)
    def _():
        zero = jnp.zeros((1, 1), jnp.float32)
        loss_ref[...] = zero
        sq_ref[...] = zero
        sp_ref[...] = zero
        sn_ref[...] = zero
        st_ref[...] = zero

    loss_ref[...] += jnp.reshape(part, (1, 1))
    sq_ref[...] += jnp.reshape(jnp.sum(q * q), (1, 1))
    sp_ref[...] += jnp.reshape(jnp.sum(pI * pI), (1, 1))
    sn_ref[...] += jnp.reshape(jnp.sum(nI * nI), (1, 1))
    st_ref[...] += jnp.reshape(jnp.sum(tb * tb), (1, 1))


_SCALAR = jax.ShapeDtypeStruct((1, 1), jnp.float32)

_TC_IN_SPECS = [
        pl.BlockSpec((_BB, _IMG_FEA), lambda i: (i, 0)),
        pl.BlockSpec((_BB, _IMG_FEA), lambda i: (i + _GRID, 0)),
        pl.BlockSpec((_BB, _IMG_FEA), lambda i: (i + 2 * _GRID, 0)),
        pl.BlockSpec((_NHID, _IMG_FEA), lambda i: (0, 0)),
        pl.BlockSpec((1, _NHID), lambda i: (0, 0)),
        pl.BlockSpec((1, _NHID), lambda i: (0, 0)),
        pl.BlockSpec((1, 1), lambda i: (0, 0)),
        pl.BlockSpec((_BB * _P, _NHID), lambda i: (i, 0)),
        pl.BlockSpec((_BB * _P, _NHID), lambda i: (i + _GRID, 0)),
        pl.BlockSpec((_BB, _P), lambda i: (i, 0)),
        pl.BlockSpec((_BB, _P), lambda i: (i, 0)),
        pl.BlockSpec((1, _TABF), lambda i: (0, i)),
]

_tc_finish = pl.pallas_call(
    _tc_body,
    grid=(_GRID,),
    in_specs=_TC_IN_SPECS,
    out_specs=[pl.BlockSpec((1, 1), lambda i: (0, 0))] * 5,
    out_shape=[_SCALAR] * 5,
)


def kernel(qry_id, pos_id, neg_id, pos_path, pos_mask, pos_leafnodeMask,
           neg_path, neg_mask, neg_leafnodeMask, img_features, imageW_w,
           imageW_b, meta_table, h_att_w, h_att_b):
    f32 = jnp.float32
    ids = jnp.concatenate([qry_id, pos_id, neg_id], axis=0)[:, 0].astype(jnp.int32)
    paths = jnp.concatenate(
        [pos_path.reshape(-1), neg_path.reshape(-1)]).astype(jnp.int32)
    masks = jnp.concatenate(
        [pos_mask.reshape(-1), neg_mask.reshape(-1)]).astype(f32)

    img_rows, res = _sc_gather_combine()(
        img_features.astype(f32), ids, meta_table.astype(f32), paths, masks)

    loss, sq, sp, sn, st = _tc_finish(
        img_rows, img_rows, img_rows,
        imageW_w.astype(f32), imageW_b.reshape(1, _NHID).astype(f32),
        h_att_w.astype(f32), h_att_b.reshape(1, 1).astype(f32),
        res, res,
        pos_leafnodeMask.astype(f32), neg_leafnodeMask.astype(f32),
        meta_table.reshape(1, -1).astype(f32))

    return (loss[0, 0] + _REG * (jnp.sqrt(st[0, 0]) + jnp.sqrt(sq[0, 0])
                                 + jnp.sqrt(sp[0, 0]) + jnp.sqrt(sn[0, 0])))


# speedup vs baseline: 6.9646x; 1.0263x over previous
import functools

import jax
import jax.numpy as jnp
from jax import lax
from jax.experimental import pallas as pl
from jax.experimental.pallas import tpu as pltpu
from jax.experimental.pallas import tpu_sc as plsc

_B, _P, _L = 1024, 20, 8
_NHID = 128
_IMG_FEA = 512
_META_VOCAB = 10000
_REG = 0.001

_NW = 32
_IMG_N = 3 * _B
_IMG_PER = _IMG_N // _NW
_NT = 2 * _B * _P
_TASKS_PER = _NT // _NW
_CH = 32
_NCHUNK = _TASKS_PER // _CH


def _sc_body(img_feat, img_ids, table, paths, masks,
                       img_out, res_out,
                       iidx_v, irows_v, idx_v, rows_v, m_v, out_v, sem, osem):
    wid = lax.axis_index("s") * 2 + lax.axis_index("c")

    ibase = wid * _IMG_PER
    pltpu.sync_copy(img_ids.at[pl.ds(ibase, _IMG_PER)], iidx_v)
    for h in range(2):
        pltpu.async_copy(
            img_feat.at[iidx_v.at[pl.ds(h * _IMG_PER // 2, _IMG_PER // 2)]],
            irows_v, sem).wait()
        pltpu.sync_copy(
            irows_v, img_out.at[pl.ds(ibase + h * _IMG_PER // 2, _IMG_PER // 2)])

    tbase = wid * _TASKS_PER
    ones = jnp.ones((16,), jnp.float32)
    _R = _CH * _L

    pltpu.sync_copy(masks.at[pl.ds(tbase * _L, _TASKS_PER * _L)], m_v)

    def fire(g, slot):
        base = tbase + g * _CH
        pltpu.sync_copy(paths.at[pl.ds(base * _L, _R)],
                        idx_v.at[pl.ds(slot * _R, _R)])
        for h in range(2):
            pltpu.async_copy(
                table.at[idx_v.at[pl.ds(slot * _R + h * 128, 128)]],
                rows_v.at[pl.ds(slot * _R + h * 128, 128)], sem)

    def run_tasks(roff, ooff, moff):
        @plsc.parallel_loop(0, _CH // 2, 1, unroll=4)
        def _(tp):
            mv = m_v[pl.ds(moff + tp * 16, 16)]
            for half in range(2):
                t = tp * 2 + half
                ms = [jnp.full((16,), mv[half * _L + l], jnp.float32)
                      for l in range(_L)]
                c0 = [ms[2 * i] for i in range(4)]
                c1 = [ms[2 * i + 1] * ms[2 * i + 1] for i in range(4)]
                c2 = [(ones - ms[2 * i]) + (ms[2 * i + 1] - c1[i])
                      for i in range(4)]
                for c in range(_NHID // 16):
                    a = []
                    for i in range(4):
                        t0 = rows_v[roff + t * _L + 2 * i, pl.ds(c * 16, 16)]
                        t1 = rows_v[roff + t * _L + 2 * i + 1, pl.ds(c * 16, 16)]
                        a.append(t0 * c0[i] + t1 * c1[i] + c2[i])
                    out_v[ooff + t, pl.ds(c * 16, 16)] = (
                        a[1] * (a[0] + a[2]) + a[2] * a[3])

    fire(0, 0)

    def chunk_body(g, carry):
        slot = lax.rem(g, 2)
        nslot = 1 - slot
        base = tbase + g * _CH

        @pl.when(g + 1 < _NCHUNK)
        def _():
            fire(g + 1, nslot)

        @pl.when(g >= 2)
        def _():
            pltpu.make_async_copy(
                out_v.at[pl.ds(slot * _CH, _CH)],
                res_out.at[pl.ds(base, _CH)], osem).wait()

        for h in range(2):
            pltpu.make_async_copy(
                table.at[idx_v.at[pl.ds(slot * _R + h * 128, 128)]],
                rows_v.at[pl.ds(slot * _R + h * 128, 128)], sem).wait()

        run_tasks(slot * _R, slot * _CH, g * _CH * _L)
        pltpu.async_copy(out_v.at[pl.ds(slot * _CH, _CH)],
                         res_out.at[pl.ds(base, _CH)], osem)
        return carry

    lax.fori_loop(0, _NCHUNK, chunk_body, 0)

    for k in range(2):
        pltpu.make_async_copy(
            out_v.at[pl.ds(k * _CH, _CH)],
            res_out.at[pl.ds(tbase + k * _CH, _CH)], osem).wait()


@functools.cache
def _sc_gather_combine():
    return pl.kernel(
        _sc_body,
        mesh=plsc.VectorSubcoreMesh(core_axis_name="c", subcore_axis_name="s"),
        out_type=[
            jax.ShapeDtypeStruct((_IMG_N, _IMG_FEA), jnp.float32),
            jax.ShapeDtypeStruct((_NT, _NHID), jnp.float32),
        ],
        scratch_types=[
            pltpu.VMEM((_IMG_PER,), jnp.int32),
            pltpu.VMEM((_IMG_PER // 2, _IMG_FEA), jnp.float32),
            pltpu.VMEM((2 * _CH * _L,), jnp.int32),
            pltpu.VMEM((2 * _CH * _L, _NHID), jnp.float32),
            pltpu.VMEM((_TASKS_PER * _L,), jnp.float32),
            pltpu.VMEM((2 * _CH, _NHID), jnp.float32),
            pltpu.SemaphoreType.DMA,
            pltpu.SemaphoreType.DMA,
        ],
    )


_BB = 128
_GRID = _B // _BB
_TABF = _META_VOCAB * _NHID // _GRID


def _tc_body(qr, pr, nr, w, b2, hw, hb2, pres, nres, pleaf, nleaf, tabf,
             loss_ref, sq_ref, sp_ref, sn_ref, st_ref):
    i = pl.program_id(0)
    W = w[...]
    dn = (((1,), (1,)), ((), ()))
    q = lax.dot_general(qr[...], W, dn, preferred_element_type=jnp.float32) + b2[...]
    pI = lax.dot_general(pr[...], W, dn, preferred_element_type=jnp.float32) + b2[...]
    nI = lax.dot_general(nr[...], W, dn, preferred_element_type=jnp.float32) + b2[...]
    hwv = hw[...]
    hb = hb2[0, 0]

    def side(res_ref, leaf_ref, item):
        r = res_ref[...].reshape(_BB, _P, _NHID)
        ssq = jnp.sum(r * r, axis=-1, keepdims=True)
        pe = r / jnp.maximum(jnp.sqrt(ssq), 1e-12)
        uim = q * item
        uis = q - item
        amp = jnp.sum(uim * hwv, axis=-1)
        v = uis * hwv
        wgt = amp[:, None] - jnp.sum(v[:, None, :] * pe, axis=-1) + hb
        wgt = wgt * jax.nn.sigmoid(leaf_ref[...] * 2.0)
        wgt = wgt - jnp.max(wgt, axis=-1, keepdims=True)
        e = jnp.exp(wgt)
        wsm = e / jnp.sum(e, axis=-1, keepdims=True)
        pool = jnp.sum(pe * wsm[..., None], axis=1)
        return jnp.sum(q * item + (item - q) * pool, axis=-1)

    ps = side(pres, pleaf, pI)
    ns = side(nres, nleaf, nI)
    part = jnp.sum(jnp.log1p(jnp.exp(ns - ps)))
    tb = tabf[...]

    @pl.when(i == 0)
    def _():
        zero = jnp.zeros((1, 1), jnp.float32)
        loss_ref[...] = zero
        sq_ref[...] = zero
        sp_ref[...] = zero
        sn_ref[...] = zero
        st_ref[...] = zero

    loss_ref[...] += jnp.reshape(part, (1, 1))
    sq_ref[...] += jnp.reshape(jnp.sum(q * q), (1, 1))
    sp_ref[...] += jnp.reshape(jnp.sum(pI * pI), (1, 1))
    sn_ref[...] += jnp.reshape(jnp.sum(nI * nI), (1, 1))
    st_ref[...] += jnp.reshape(jnp.sum(tb * tb), (1, 1))


_SCALAR = jax.ShapeDtypeStruct((1, 1), jnp.float32)

_TC_IN_SPECS = [
        pl.BlockSpec((_BB, _IMG_FEA), lambda i: (i, 0)),
        pl.BlockSpec((_BB, _IMG_FEA), lambda i: (i + _GRID, 0)),
        pl.BlockSpec((_BB, _IMG_FEA), lambda i: (i + 2 * _GRID, 0)),
        pl.BlockSpec((_NHID, _IMG_FEA), lambda i: (0, 0)),
        pl.BlockSpec((1, _NHID), lambda i: (0, 0)),
        pl.BlockSpec((1, _NHID), lambda i: (0, 0)),
        pl.BlockSpec((1, 1), lambda i: (0, 0)),
        pl.BlockSpec((_BB * _P, _NHID), lambda i: (i, 0)),
        pl.BlockSpec((_BB * _P, _NHID), lambda i: (i + _GRID, 0)),
        pl.BlockSpec((_BB, _P), lambda i: (i, 0)),
        pl.BlockSpec((_BB, _P), lambda i: (i, 0)),
        pl.BlockSpec((1, _TABF), lambda i: (0, i)),
]

_tc_finish = pl.pallas_call(
    _tc_body,
    grid=(_GRID,),
    in_specs=_TC_IN_SPECS,
    out_specs=[pl.BlockSpec((1, 1), lambda i: (0, 0))] * 5,
    out_shape=[_SCALAR] * 5,
)


def kernel(qry_id, pos_id, neg_id, pos_path, pos_mask, pos_leafnodeMask,
           neg_path, neg_mask, neg_leafnodeMask, img_features, imageW_w,
           imageW_b, meta_table, h_att_w, h_att_b):
    f32 = jnp.float32
    ids = jnp.concatenate([qry_id, pos_id, neg_id], axis=0)[:, 0].astype(jnp.int32)
    paths = jnp.concatenate(
        [pos_path.reshape(-1), neg_path.reshape(-1)]).astype(jnp.int32)
    masks = jnp.concatenate(
        [pos_mask.reshape(-1), neg_mask.reshape(-1)]).astype(f32)

    img_rows, res = _sc_gather_combine()(
        img_features.astype(f32), ids, meta_table.astype(f32), paths, masks)

    loss, sq, sp, sn, st = _tc_finish(
        img_rows, img_rows, img_rows,
        imageW_w.astype(f32), imageW_b.reshape(1, _NHID).astype(f32),
        h_att_w.astype(f32), h_att_b.reshape(1, 1).astype(f32),
        res, res,
        pos_leafnodeMask.astype(f32), neg_leafnodeMask.astype(f32),
        meta_table.reshape(1, -1).astype(f32))

    return (loss[0, 0] + _REG * (jnp.sqrt(st[0, 0]) + jnp.sqrt(sq[0, 0])
                                 + jnp.sqrt(sp[0, 0]) + jnp.sqrt(sn[0, 0])))


# speedup vs baseline: 7.0521x; 1.0126x over previous
import functools

import jax
import jax.numpy as jnp
from jax import lax
from jax.experimental import pallas as pl
from jax.experimental.pallas import tpu as pltpu
from jax.experimental.pallas import tpu_sc as plsc

_B, _P, _L = 1024, 20, 8
_NHID = 128
_IMG_FEA = 512
_META_VOCAB = 10000
_REG = 0.001

_NW = 32
_IMG_N = 3 * _B
_IMG_PER = _IMG_N // _NW
_NT = 2 * _B * _P
_TASKS_PER = _NT // _NW
_CH = 32
_NCHUNK = _TASKS_PER // _CH


def _sc_body(img_feat, img_ids, table, paths, masks,
                       img_out, res_out,
                       iidx_v, irows_v, idx_v, rows_v, m_v, out_v, sem, osem):
    wid = lax.axis_index("s") * 2 + lax.axis_index("c")

    ibase = wid * _IMG_PER
    pltpu.sync_copy(img_ids.at[pl.ds(ibase, _IMG_PER)], iidx_v)
    for h in range(2):
        pltpu.async_copy(
            img_feat.at[iidx_v.at[pl.ds(h * _IMG_PER // 2, _IMG_PER // 2)]],
            irows_v, sem).wait()
        pltpu.sync_copy(
            irows_v, img_out.at[pl.ds(ibase + h * _IMG_PER // 2, _IMG_PER // 2)])

    tbase = wid * _TASKS_PER
    ones = jnp.ones((16,), jnp.float32)
    _R = _CH * _L

    pltpu.sync_copy(masks.at[pl.ds(tbase * _L, _TASKS_PER * _L)], m_v)

    def fire(g, slot):
        base = tbase + g * _CH
        pltpu.sync_copy(paths.at[pl.ds(base * _L, _R)],
                        idx_v.at[pl.ds(slot * _R, _R)])
        for h in range(2):
            pltpu.async_copy(
                table.at[idx_v.at[pl.ds(slot * _R + h * 128, 128)]],
                rows_v.at[pl.ds(slot * _R + h * 128, 128)], sem)

    def run_tasks(roff, ooff, moff):
        @plsc.parallel_loop(0, _CH // 2, 1, unroll=4)
        def _(tp):
            mv = m_v[pl.ds(moff + tp * 16, 16)]
            for half in range(2):
                t = tp * 2 + half
                ms = [jnp.full((16,), mv[half * _L + l], jnp.float32)
                      for l in range(_L)]
                c0 = [ms[2 * i] for i in range(4)]
                c1 = [ms[2 * i + 1] * ms[2 * i + 1] for i in range(4)]
                c2 = [(ones - ms[2 * i]) + (ms[2 * i + 1] - c1[i])
                      for i in range(4)]
                for c in range(_NHID // 16):
                    a = []
                    for i in range(4):
                        t0 = rows_v[roff + t * _L + 2 * i, pl.ds(c * 16, 16)]
                        t1 = rows_v[roff + t * _L + 2 * i + 1, pl.ds(c * 16, 16)]
                        a.append(t0 * c0[i] + t1 * c1[i] + c2[i])
                    out_v[ooff + t, pl.ds(c * 16, 16)] = (
                        a[1] * (a[0] + a[2]) + a[2] * a[3])

    fire(0, 0)

    def chunk_body(g, carry):
        slot = lax.rem(g, 2)
        nslot = 1 - slot
        base = tbase + g * _CH

        @pl.when(g + 1 < _NCHUNK)
        def _():
            fire(g + 1, nslot)

        @pl.when(g >= 2)
        def _():
            pltpu.make_async_copy(
                out_v.at[pl.ds(slot * _CH, _CH)],
                res_out.at[pl.ds(base, _CH)], osem).wait()

        for h in range(2):
            pltpu.make_async_copy(
                table.at[idx_v.at[pl.ds(slot * _R + h * 128, 128)]],
                rows_v.at[pl.ds(slot * _R + h * 128, 128)], sem).wait()

        run_tasks(slot * _R, slot * _CH, g * _CH * _L)
        pltpu.async_copy(out_v.at[pl.ds(slot * _CH, _CH)],
                         res_out.at[pl.ds(base, _CH)], osem)
        return carry

    lax.fori_loop(0, _NCHUNK, chunk_body, 0)

    for k in range(2):
        pltpu.make_async_copy(
            out_v.at[pl.ds(k * _CH, _CH)],
            res_out.at[pl.ds(tbase + k * _CH, _CH)], osem).wait()


@functools.cache
def _sc_gather_combine():
    return pl.kernel(
        _sc_body,
        mesh=plsc.VectorSubcoreMesh(core_axis_name="c", subcore_axis_name="s"),
        out_type=[
            jax.ShapeDtypeStruct((_IMG_N, _IMG_FEA), jnp.float32),
            jax.ShapeDtypeStruct((_NT, _NHID), jnp.float32),
        ],
        scratch_types=[
            pltpu.VMEM((_IMG_PER,), jnp.int32),
            pltpu.VMEM((_IMG_PER // 2, _IMG_FEA), jnp.float32),
            pltpu.VMEM((2 * _CH * _L,), jnp.int32),
            pltpu.VMEM((2 * _CH * _L, _NHID), jnp.float32),
            pltpu.VMEM((_TASKS_PER * _L,), jnp.float32),
            pltpu.VMEM((2 * _CH, _NHID), jnp.float32),
            pltpu.SemaphoreType.DMA,
            pltpu.SemaphoreType.DMA,
        ],
    )


_BB = 128
_GRID = _B // _BB
_TABF = _META_VOCAB * _NHID // _GRID


def _tc_body(qr, pr, nr, w, b2, hw, hb2, pres, nres, pleaf, nleaf, tabf,
             loss_ref, sq_ref, sp_ref, sn_ref, st_ref):
    i = pl.program_id(0)
    W = w[...]
    dn = (((1,), (1,)), ((), ()))
    q = lax.dot_general(qr[...], W, dn, preferred_element_type=jnp.float32) + b2[...]
    pI = lax.dot_general(pr[...], W, dn, preferred_element_type=jnp.float32) + b2[...]
    nI = lax.dot_general(nr[...], W, dn, preferred_element_type=jnp.float32) + b2[...]
    hwv = hw[...]
    hb = hb2[0, 0]

    def side(res_ref, leaf_ref, item):
        r = res_ref[...].reshape(_BB, _P, _NHID)
        v = (q - item) * hwv
        u2 = item - q
        S1 = jnp.sum(r * r, axis=-1)
        S2 = jnp.sum(v[:, None, :] * r, axis=-1)
        S3 = jnp.sum(u2[:, None, :] * r, axis=-1)
        rsq = 1.0 / jnp.maximum(jnp.sqrt(S1), 1e-12)
        amp = jnp.sum((q * item) * hwv, axis=-1)
        wgt = (amp[:, None] - S2 * rsq + hb) * jax.nn.sigmoid(leaf_ref[...] * 2.0)
        wgt = wgt - jnp.max(wgt, axis=-1, keepdims=True)
        e = jnp.exp(wgt)
        wsm = e / jnp.sum(e, axis=-1, keepdims=True)
        return jnp.sum(q * item, axis=-1) + jnp.sum(wsm * S3 * rsq, axis=-1)

    ps = side(pres, pleaf, pI)
    ns = side(nres, nleaf, nI)
    part = jnp.sum(jnp.log1p(jnp.exp(ns - ps)))
    tb = tabf[...]

    @pl.when(i == 0)
    def _():
        zero = jnp.zeros((1, 1), jnp.float32)
        loss_ref[...] = zero
        sq_ref[...] = zero
        sp_ref[...] = zero
        sn_ref[...] = zero
        st_ref[...] = zero

    loss_ref[...] += jnp.reshape(part, (1, 1))
    sq_ref[...] += jnp.reshape(jnp.sum(q * q), (1, 1))
    sp_ref[...] += jnp.reshape(jnp.sum(pI * pI), (1, 1))
    sn_ref[...] += jnp.reshape(jnp.sum(nI * nI), (1, 1))
    st_ref[...] += jnp.reshape(jnp.sum(tb * tb), (1, 1))


_SCALAR = jax.ShapeDtypeStruct((1, 1), jnp.float32)

_TC_IN_SPECS = [
        pl.BlockSpec((_BB, _IMG_FEA), lambda i: (i, 0)),
        pl.BlockSpec((_BB, _IMG_FEA), lambda i: (i + _GRID, 0)),
        pl.BlockSpec((_BB, _IMG_FEA), lambda i: (i + 2 * _GRID, 0)),
        pl.BlockSpec((_NHID, _IMG_FEA), lambda i: (0, 0)),
        pl.BlockSpec((1, _NHID), lambda i: (0, 0)),
        pl.BlockSpec((1, _NHID), lambda i: (0, 0)),
        pl.BlockSpec((1, 1), lambda i: (0, 0)),
        pl.BlockSpec((_BB * _P, _NHID), lambda i: (i, 0)),
        pl.BlockSpec((_BB * _P, _NHID), lambda i: (i + _GRID, 0)),
        pl.BlockSpec((_BB, _P), lambda i: (i, 0)),
        pl.BlockSpec((_BB, _P), lambda i: (i, 0)),
        pl.BlockSpec((1, _TABF), lambda i: (0, i)),
]

_tc_finish = pl.pallas_call(
    _tc_body,
    grid=(_GRID,),
    in_specs=_TC_IN_SPECS,
    out_specs=[pl.BlockSpec((1, 1), lambda i: (0, 0))] * 5,
    out_shape=[_SCALAR] * 5,
)


def kernel(qry_id, pos_id, neg_id, pos_path, pos_mask, pos_leafnodeMask,
           neg_path, neg_mask, neg_leafnodeMask, img_features, imageW_w,
           imageW_b, meta_table, h_att_w, h_att_b):
    f32 = jnp.float32
    ids = jnp.concatenate([qry_id, pos_id, neg_id], axis=0)[:, 0].astype(jnp.int32)
    paths = jnp.concatenate(
        [pos_path.reshape(-1), neg_path.reshape(-1)]).astype(jnp.int32)
    masks = jnp.concatenate(
        [pos_mask.reshape(-1), neg_mask.reshape(-1)]).astype(f32)

    img_rows, res = _sc_gather_combine()(
        img_features.astype(f32), ids, meta_table.astype(f32), paths, masks)

    loss, sq, sp, sn, st = _tc_finish(
        img_rows, img_rows, img_rows,
        imageW_w.astype(f32), imageW_b.reshape(1, _NHID).astype(f32),
        h_att_w.astype(f32), h_att_b.reshape(1, 1).astype(f32),
        res, res,
        pos_leafnodeMask.astype(f32), neg_leafnodeMask.astype(f32),
        meta_table.reshape(1, -1).astype(f32))

    return (loss[0, 0] + _REG * (jnp.sqrt(st[0, 0]) + jnp.sqrt(sq[0, 0])
                                 + jnp.sqrt(sp[0, 0]) + jnp.sqrt(sn[0, 0])))
